# Initial kernel scaffold; baseline (speedup 1.0000x reference)
import jax
import jax.numpy as jnp
from jax.experimental import pallas as pl


def kernel(x_rna, edge_index_rna, e_rna, x_atac, edge_index_atac, e_atac, x_cell, W1_rna, b1_rna, W2_rna, b2_rna, Wu_rna, bu_rna, W1_atac, b1_atac, W2_atac, b2_atac, Wu_atac, bu_atac, Wc, bc):
    raise NotImplementedError("write your pallas kernel here")


# speedup vs baseline: 3.0608x; 3.0608x over previous
import functools

import jax
import jax.numpy as jnp
from jax import lax
from jax.experimental import pallas as pl
from jax.experimental.pallas import tpu as pltpu
from jax.experimental.pallas import tpu_sc as plsc

N, E, D, DE, H = 10000, 320000, 128, 16, 128

_NC, _NS = 2, 16
_NW = _NC * _NS
_CHUNK = 80
_EPT = (2 * E) // _NW
_ITERS = _EPT // _CHUNK
_NBUF = 2

_EPT_S = E // _NS
_SITERS = _EPT_S // _CHUNK

_NBLK = 2000
_R = 1000
_MBLK = E // _R


def _proj_body(x_ref, w_ref, o_ref):
    o_ref[...] = jnp.dot(x_ref[0], w_ref[0], preferred_element_type=jnp.float32)


def _node_proj(xs, Ws):
    nb = N // _NBLK
    return pl.pallas_call(
        _proj_body,
        grid=(4, nb),
        in_specs=[
            pl.BlockSpec((1, _NBLK, D), lambda j, i: (j // 2, i, 0)),
            pl.BlockSpec((1, D, H), lambda j, i: (j, 0, 0)),
        ],
        out_specs=pl.BlockSpec((_NBLK, H), lambda j, i: (j * (N // _NBLK) + i, 0)),
        out_shape=jax.ShapeDtypeStruct((4 * N, H), jnp.float32),
    )(xs, Ws)


def _sc_gather_body(*refs):
    T_hbm, idxp_hbm, idxq_hbm = refs[0:3]
    gp_hbm, gq_hbm = refs[3:5]
    idxp_v, idxq_v = refs[5:7]
    bufp = refs[7:7 + _NBUF]
    bufq = refs[7 + _NBUF:7 + 2 * _NBUF]
    semp = refs[7 + 2 * _NBUF:7 + 3 * _NBUF]
    semq = refs[7 + 3 * _NBUF:7 + 4 * _NBUF]

    c = lax.axis_index("c")
    s = lax.axis_index("s")
    w = s * _NC + c
    base = w * _EPT

    pltpu.sync_copy(idxp_hbm.at[w], idxp_v)
    pltpu.sync_copy(idxq_hbm.at[w], idxq_v)

    for b in range(_NBUF):
        pltpu.async_copy(T_hbm.at[idxp_v.at[b]], bufp[b], semp[b])
        pltpu.async_copy(T_hbm.at[idxq_v.at[b]], bufq[b], semq[b])

    def step(i, carry):
        for b in range(_NBUF):
            ib = i * _NBUF + b
            off = base + ib * _CHUNK
            pltpu.make_async_copy(T_hbm.at[idxp_v.at[ib]], bufp[b], semp[b]).wait()
            pltpu.sync_copy(bufp[b], gp_hbm.at[pl.ds(off, _CHUNK)])
            pltpu.make_async_copy(T_hbm.at[idxq_v.at[ib]], bufq[b], semq[b]).wait()
            pltpu.sync_copy(bufq[b], gq_hbm.at[pl.ds(off, _CHUNK)])
            nxt = ib + _NBUF

            @pl.when(nxt < _ITERS)
            def _():
                pltpu.async_copy(T_hbm.at[idxp_v.at[nxt]], bufp[b], semp[b])
                pltpu.async_copy(T_hbm.at[idxq_v.at[nxt]], bufq[b], semq[b])

        return carry

    lax.fori_loop(0, _ITERS // _NBUF, step, 0)


def _sc_gather(T, idxP, idxQ):
    mesh = plsc.VectorSubcoreMesh(core_axis_name="c", subcore_axis_name="s")
    scratch = (
        [pltpu.VMEM((_ITERS, _CHUNK), jnp.int32)] * 2
        + [pltpu.VMEM((_CHUNK, H), jnp.float32)] * (2 * _NBUF)
        + [pltpu.SemaphoreType.DMA] * (2 * _NBUF)
    )
    out = jax.ShapeDtypeStruct((2 * E, H), jnp.float32)
    return pl.kernel(
        _sc_gather_body,
        out_type=[out, out],
        mesh=mesh,
        scratch_types=scratch,
    )(T, idxP, idxQ)


def _mlp_body(gp_ref, gq_ref, e_ref, w1e_ref, b1_ref, w2_ref, b2_ref, o_ref):
    g = (gp_ref[...] + gq_ref[...]
         + jnp.dot(e_ref[...], w1e_ref[0], preferred_element_type=jnp.float32)
         + b1_ref[0])
    h1 = jax.nn.silu(g)
    h2 = jax.nn.silu(jnp.dot(h1, w2_ref[0], preferred_element_type=jnp.float32)
                     + b2_ref[0])
    o_ref[...] = jax.nn.silu(h2)


def _edge_mlp(gp, gq, e_all, w1e, b1s, w2s, b2s):
    return pl.pallas_call(
        _mlp_body,
        grid=(2 * E // _R,),
        in_specs=[
            pl.BlockSpec((_R, H), lambda i: (i, 0)),
            pl.BlockSpec((_R, H), lambda i: (i, 0)),
            pl.BlockSpec((_R, DE), lambda i: (i, 0)),
            pl.BlockSpec((1, DE, H), lambda i: (i // _MBLK, 0, 0)),
            pl.BlockSpec((1, 1, H), lambda i: (i // _MBLK, 0, 0)),
            pl.BlockSpec((1, H, H), lambda i: (i // _MBLK, 0, 0)),
            pl.BlockSpec((1, 1, H), lambda i: (i // _MBLK, 0, 0)),
        ],
        out_specs=pl.BlockSpec((_R, H), lambda i: (i, 0)),
        out_shape=jax.ShapeDtypeStruct((2 * E, H), jnp.float32),
    )(gp, gq, e_all, w1e, b1s, w2s, b2s)


def _sc_scatter_body(*refs):
    h_hbm, dst_hbm, zeros_hbm = refs[0:3]
    out_hbm = refs[3]
    idxb = refs[4:4 + _NBUF]
    rows = refs[4 + _NBUF:4 + 2 * _NBUF]
    semi = refs[4 + 2 * _NBUF:4 + 3 * _NBUF]
    semr = refs[4 + 3 * _NBUF:4 + 4 * _NBUF]
    acc = refs[4 + 4 * _NBUF]

    c = lax.axis_index("c")
    s = lax.axis_index("s")
    base = c * E + s * _EPT_S

    @pl.when(s < 15)
    def _():
        pltpu.sync_copy(zeros_hbm.at[pl.ds(s * 632, 632)], acc.at[pl.ds(s * 632, 632)])

    @pl.when(s == 15)
    def _():
        pltpu.sync_copy(zeros_hbm.at[pl.ds(9480, 520)], acc.at[pl.ds(9480, 520)])

    plsc.subcore_barrier()

    for b in range(_NBUF):
        pltpu.async_copy(dst_hbm.at[c, s, b], idxb[b], semi[b])
        pltpu.async_copy(h_hbm.at[pl.ds(base + b * _CHUNK, _CHUNK)], rows[b], semr[b])

    def step(i, carry):
        for b in range(_NBUF):
            ib = i * _NBUF + b
            pltpu.make_async_copy(dst_hbm.at[c, s, ib], idxb[b], semi[b]).wait()
            pltpu.make_async_copy(
                h_hbm.at[pl.ds(base + ib * _CHUNK, _CHUNK)], rows[b], semr[b]).wait()
            pltpu.sync_copy(rows[b], acc.at[idxb[b]], add=True)
            nxt = ib + _NBUF

            @pl.when(nxt < _SITERS)
            def _():
                pltpu.async_copy(dst_hbm.at[c, s, nxt], idxb[b], semi[b])
                pltpu.async_copy(
                    h_hbm.at[pl.ds(base + nxt * _CHUNK, _CHUNK)], rows[b], semr[b])

        return carry

    lax.fori_loop(0, _SITERS // _NBUF, step, 0)

    plsc.subcore_barrier()

    @pl.when(s < 15)
    def _():
        pltpu.sync_copy(acc.at[pl.ds(s * 632, 632)],
                        out_hbm.at[c].at[pl.ds(s * 632, 632)])

    @pl.when(s == 15)
    def _():
        pltpu.sync_copy(acc.at[pl.ds(9480, 520)],
                        out_hbm.at[c].at[pl.ds(9480, 520)])


def _sc_scatter(h, dsts, zeros):
    mesh = plsc.VectorSubcoreMesh(core_axis_name="c", subcore_axis_name="s")
    scratch = (
        [pltpu.VMEM((_CHUNK,), jnp.int32)] * _NBUF
        + [pltpu.VMEM((_CHUNK, H), jnp.float32)] * _NBUF
        + [pltpu.SemaphoreType.DMA] * (2 * _NBUF)
        + [pltpu.VMEM_SHARED((N, H), jnp.float32)]
    )
    return pl.kernel(
        _sc_scatter_body,
        out_type=jax.ShapeDtypeStruct((2, N, H), jnp.float32),
        mesh=mesh,
        scratch_types=scratch,
    )(h, dsts, zeros)


def _upd_body(a_ref, w_ref, b_ref, o_ref):
    y = jnp.dot(a_ref[0], w_ref[0], preferred_element_type=jnp.float32) + b_ref[0]
    o_ref[0] = jnp.where(pl.program_id(0) == 2, jax.nn.silu(y), y)


def _final(A, Wst, bst):
    nb = N // _NBLK
    return pl.pallas_call(
        _upd_body,
        grid=(3, nb),
        in_specs=[
            pl.BlockSpec((1, _NBLK, H), lambda j, i: (j, i, 0)),
            pl.BlockSpec((1, H, H), lambda j, i: (j, 0, 0)),
            pl.BlockSpec((1, 1, H), lambda j, i: (j, 0, 0)),
        ],
        out_specs=pl.BlockSpec((1, _NBLK, H), lambda j, i: (j, i, 0)),
        out_shape=jax.ShapeDtypeStruct((3, N, H), jnp.float32),
    )(A, Wst, bst)


def kernel(x_rna, edge_index_rna, e_rna, x_atac, edge_index_atac, e_atac, x_cell,
           W1_rna, b1_rna, W2_rna, b2_rna, Wu_rna, bu_rna,
           W1_atac, b1_atac, W2_atac, b2_atac, Wu_atac, bu_atac,
           Wc, bc):
    src_r = edge_index_rna[0].astype(jnp.int32)
    dst_r = edge_index_rna[1].astype(jnp.int32)
    src_a = edge_index_atac[0].astype(jnp.int32)
    dst_a = edge_index_atac[1].astype(jnp.int32)

    xs = jnp.stack([x_rna, x_atac])
    Ws = jnp.stack([W1_rna[:D], W1_rna[D:2 * D], W1_atac[:D], W1_atac[D:2 * D]])
    T = _node_proj(xs, Ws)

    idxP = jnp.concatenate([dst_r, dst_a + 2 * N]).reshape(_NW, _ITERS, _CHUNK)
    idxQ = jnp.concatenate([src_r + N, src_a + 3 * N]).reshape(_NW, _ITERS, _CHUNK)
    gp, gq = _sc_gather(T, idxP, idxQ)

    e_all = jnp.concatenate([e_rna, e_atac])
    w1e = jnp.stack([W1_rna[2 * D:], W1_atac[2 * D:]])
    b1s = jnp.stack([b1_rna, b1_atac])[:, None, :]
    w2s = jnp.stack([W2_rna, W2_atac])
    b2s = jnp.stack([b2_rna, b2_atac])[:, None, :]
    h = _edge_mlp(gp, gq, e_all, w1e, b1s, w2s, b2s)

    dsts = jnp.concatenate([dst_r, dst_a]).reshape(2, _NS, _SITERS, _CHUNK)
    zeros = jnp.zeros((N, H), jnp.float32)
    aggr = _sc_scatter(h, dsts, zeros)

    A = jnp.concatenate([aggr, x_cell[None]], axis=0)
    Wst = jnp.stack([Wu_rna, Wu_atac, Wc])
    bst = jnp.stack([bu_rna, bu_atac, bc])[:, None, :]
    out = _final(A, Wst, bst)
    return out[0], out[1], out[2]
